# baseline (speedup 1.0000x reference)
import functools

import jax
import jax.numpy as jnp
import numpy as np
from jax import lax
from jax.experimental import pallas as pl
from jax.experimental.pallas import tpu as pltpu
from jax.experimental.pallas import tpu_sc as plsc

N = 10000
E = 320000
D = 128
H = 8
C = 16
DE = 16
HID = 128

NC = 2
NS = 16
HL = H // NC
HW = HL * C
NSPLIT = 4
EH = E // NSPLIT
EPS = EH // NS
B = 40
NCHUNK = EPS // B
NPAD = 10240
ROWS_PT = NPAD // NS
ACCW = 80

BN = 400
BE = 2000


def _proj_body(x_ref, wq_ref, wkv_ref, q_ref, kv_ref):
    x = x_ref[...]
    q_ref[0] = (x @ wq_ref[0]) * 0.25
    kv_ref[0] = x @ wkv_ref[0]


def _project_nodes(x, w_q2, w_kv2):
    return pl.pallas_call(
        _proj_body,
        grid=(NC, N // BN),
        in_specs=[
            pl.BlockSpec((BN, D), lambda c, i: (i, 0)),
            pl.BlockSpec((1, D, HW), lambda c, i: (c, 0, 0)),
            pl.BlockSpec((1, D, 2 * HW), lambda c, i: (c, 0, 0)),
        ],
        out_specs=[
            pl.BlockSpec((1, BN, HW), lambda c, i: (c, i, 0)),
            pl.BlockSpec((1, BN, 2 * HW), lambda c, i: (c, i, 0)),
        ],
        out_shape=[
            jax.ShapeDtypeStruct((NC, N, HW), jnp.float32),
            jax.ShapeDtypeStruct((NC, N, 2 * HW), jnp.float32),
        ],
    )(x, w_q2, w_kv2)


def _e01_body(ef_ref, we_ref, out_ref):
    out_ref[0] = jnp.tanh(ef_ref[...] @ we_ref[0]).astype(jnp.bfloat16)


def _project_edges(edge_feat, w_e01_2, s):
    nb = EH // BE
    return pl.pallas_call(
        _e01_body,
        grid=(NC, nb),
        in_specs=[
            pl.BlockSpec((BE, DE), lambda c, i: (s * nb + i, 0)),
            pl.BlockSpec((1, DE, 2 * HW), lambda c, i: (c, 0, 0)),
        ],
        out_specs=pl.BlockSpec((1, BE, 2 * HW), lambda c, i: (c, i, 0)),
        out_shape=jax.ShapeDtypeStruct((NC, EH, 2 * HW), jnp.bfloat16),
    )(edge_feat, w_e01_2)


def _sc_edge_body(sidx_hbm, didx_hbm, q_hbm, kv_hbm, e01_hbm,
                  zeros_hbm, out_hbm, sidx_v, didx_v, qi_v, kv_v,
                  e01_v, msg_v, acc_sh, gsem0, gsem1, ssem0, ssem1):
    cid = lax.axis_index("c")
    sid = lax.axis_index("s")
    gsem = (gsem0, gsem1)
    ssem = (ssem0, ssem1)
    q_t = q_hbm.at[cid]
    kv_t = kv_hbm.at[cid]

    cpi0 = pltpu.async_copy(sidx_hbm.at[sid], sidx_v, gsem[0])
    cpi1 = pltpu.async_copy(didx_hbm.at[sid], didx_v, gsem[0])
    pltpu.sync_copy(zeros_hbm, acc_sh.at[pl.ds(sid * ROWS_PT, ROWS_PT)])
    cpi0.wait()
    cpi1.wait()
    plsc.subcore_barrier()

    ebase = sid * EPS
    iota = lax.iota(jnp.int32, 16)
    _gdn = lax.GatherDimensionNumbers(offset_dims=(),
                                      collapsed_slice_dims=(0,),
                                      start_index_map=(0,))

    def _lanes(x, idx):
        return lax.gather(x, idx.reshape(16, 1), _gdn, (1,),
                          mode=lax.GatherScatterMode.PROMISE_IN_BOUNDS,
                          unique_indices=True)

    def prefetch(j, b):
        eb = ebase + j * B
        cp0 = pltpu.async_copy(q_t.at[didx_v.at[j]], qi_v.at[b], gsem[b])
        cp1 = pltpu.async_copy(kv_t.at[sidx_v.at[j]], kv_v.at[b], gsem[b])
        cp2 = pltpu.async_copy(e01_hbm.at[cid, pl.ds(eb, B)], e01_v.at[b],
                               gsem[b])
        return cp0, cp1, cp2

    def wait_prefetch(b):
        pltpu.make_async_copy(q_t.at[didx_v.at[0]], qi_v.at[b],
                              gsem[b]).wait()
        pltpu.make_async_copy(kv_t.at[sidx_v.at[0]], kv_v.at[b],
                              gsem[b]).wait()
        pltpu.make_async_copy(e01_hbm.at[cid, pl.ds(ebase, B)], e01_v.at[b],
                              gsem[b]).wait()

    def wait_scatter(b):
        pltpu.make_async_copy(msg_v.at[b], acc_sh.at[didx_v.at[0]],
                              ssem[b]).wait()

    def compute(j, b, first):
        wait_prefetch(b)
        qi = qi_v.at[b]
        kv = kv_v.at[b]
        e01 = e01_v.at[b]
        msg = msg_v.at[b]

        @pl.when(jnp.logical_not(first))
        def _():
            wait_scatter(b)

        perm8 = iota ^ 8
        perm4 = iota ^ 4
        lt4 = iota < 4

        def edge(i, exv_prev):
            ia = jnp.minimum(i, B - 1)
            im = jnp.maximum(i - 1, 0)
            qs = [qi[ia, pl.ds(16 * k, 16)] for k in range(4)]
            ks = [kv[ia, pl.ds(16 * k, 16)] for k in range(4)]
            e0s = []
            for m in range(2):
                a, bb = plsc.unpack(e01[ia, pl.ds(32 * m, 32)],
                                    format=plsc.PackFormat.INTERLEAVED)
                e0s.extend([a, bb])
            vs = [kv[im, pl.ds(HW + 16 * k, 16)] for k in range(4)]
            e1s = []
            for m in range(2):
                a, bb = plsc.unpack(e01[im, pl.ds(HW + 32 * m, 32)],
                                    format=plsc.PackFormat.INTERLEAVED)
                e1s.extend([a, bb])
            ts = [qs[k] * ks[k] * e0s[k] for k in range(4)]
            ms = [vs[k] * e1s[k] * exv_prev for k in range(4)]
            t = (ts[0] + ts[1]) + (ts[2] + ts[3])
            t = t + _lanes(t, perm8)
            t = t + _lanes(t, perm4)
            exv = jnp.exp(t)
            for k in range(4):
                msg[im, pl.ds(16 * k, 16)] = ms[k]
            msg[im, pl.ds(HW, 16)] = jnp.where(lt4, exv_prev, 0.0)
            return exv

        lax.fori_loop(0, B + 1, edge, jnp.zeros((16,), jnp.float32))
        pltpu.async_copy(msg, acc_sh.at[didx_v.at[j]], ssem[b], add=True)

    prefetch(0, 0)

    def outer(t, carry):
        j0 = 2 * t
        prefetch(j0 + 1, 1)
        compute(j0, 0, t == 0)
        prefetch(jnp.minimum(j0 + 2, NCHUNK - 1), 0)
        compute(j0 + 1, 1, t == 0)
        return carry

    lax.fori_loop(0, (NCHUNK - 1) // 2, outer, 0)
    compute(NCHUNK - 1, 0, False)
    wait_scatter(1)
    wait_scatter(0)
    plsc.subcore_barrier()

    pltpu.sync_copy(acc_sh.at[pl.ds(sid * ROWS_PT, ROWS_PT)],
                    out_hbm.at[cid, pl.ds(sid * ROWS_PT, ROWS_PT)])


def _sc_edge(sidx, didx, q2, kv2, e01_2, zeros):
    mesh = plsc.VectorSubcoreMesh(core_axis_name="c", subcore_axis_name="s")
    f = functools.partial(
        pl.kernel,
        out_type=jax.ShapeDtypeStruct((NC, NPAD, ACCW), jnp.float32),
        mesh=mesh,
        compiler_params=pltpu.CompilerParams(use_tc_tiling_on_sc=False,
                                             needs_layout_passes=False),
        scratch_types=[
            pltpu.VMEM((NCHUNK, B), jnp.int32),
            pltpu.VMEM((NCHUNK, B), jnp.int32),
            pltpu.VMEM((2, B, HW), jnp.float32),
            pltpu.VMEM((2, B, 2 * HW), jnp.float32),
            pltpu.VMEM((2, B, 2 * HW), jnp.bfloat16),
            pltpu.VMEM((2, B, ACCW), jnp.float32),
            pltpu.VMEM_SHARED((NPAD, ACCW), jnp.float32),
            pltpu.SemaphoreType.DMA,
            pltpu.SemaphoreType.DMA,
            pltpu.SemaphoreType.DMA,
            pltpu.SemaphoreType.DMA,
        ],
    )(_sc_edge_body)
    return f(sidx, didx, q2, kv2, e01_2, zeros)


def _mlp_body(*refs):
    acc_refs = refs[:2 * NSPLIT]
    (x_ref, sa_ref, sb_ref, w1a_ref, w1b_ref, b1_ref, g1_ref, be1_ref,
     w2_ref, b2_ref, out_ref) = refs[2 * NSPLIT:]
    acca = acc_refs[0][0]
    accb = acc_refs[1][0]
    for s in range(1, NSPLIT):
        acca = acca + acc_refs[2 * s][0]
        accb = accb + acc_refs[2 * s + 1][0]
    att = jnp.concatenate([acca[:, :HW], accb[:, :HW]], axis=1)
    den = acca[:, HW:] @ sa_ref[...] + accb[:, HW:] @ sb_ref[...]
    att = att / (den + 1e-16)
    h = att @ w1a_ref[...] + x_ref[...] @ w1b_ref[...] + b1_ref[...]
    mu = jnp.mean(h, axis=1, keepdims=True)
    var = jnp.mean((h - mu) ** 2, axis=1, keepdims=True)
    h = (h - mu) / jnp.sqrt(var + 1e-5) * g1_ref[...] + be1_ref[...]
    h = jnp.maximum(h, 0.0)
    out_ref[...] = h @ w2_ref[...] + b2_ref[...]


def _mlp(accs, x, sa, sb, w1a, w1b, b1, g1, be1, w2, b2):
    full = lambda shape: pl.BlockSpec(shape, lambda i: (0,) * len(shape))
    acc_specs = []
    acc_args = []
    for a in accs:
        acc_specs.append(pl.BlockSpec((1, BN, ACCW), lambda i: (0, i, 0)))
        acc_specs.append(pl.BlockSpec((1, BN, ACCW), lambda i: (1, i, 0)))
        acc_args.extend([a, a])
    return pl.pallas_call(
        _mlp_body,
        grid=(N // BN,),
        in_specs=acc_specs + [
            pl.BlockSpec((BN, D), lambda i: (i, 0)),
            full((16, 128)),
            full((16, 128)),
            full((HID, HID)),
            full((HID, HID)),
            full((1, HID)),
            full((1, HID)),
            full((1, HID)),
            full((HID, HID)),
            full((1, HID)),
        ],
        out_specs=pl.BlockSpec((BN, HID), lambda i: (i, 0)),
        out_shape=jax.ShapeDtypeStruct((N, HID), jnp.float32),
    )(*acc_args, x, sa, sb, w1a, w1b, b1, g1, be1, w2, b2)


def _perm_half(w, c):
    return w.reshape(-1, H, C)[:, HL * c:HL * (c + 1), :].transpose(
        0, 2, 1).reshape(-1, HW)


def kernel(x, edge_feat, edge_index, W_q, W_k, W_v, W_e0, W_e1, W1, b1, g1,
           be1, W2, b2):
    w_q2 = jnp.stack([_perm_half(W_q, c) for c in range(NC)])
    w_kv2 = jnp.stack([
        jnp.concatenate([_perm_half(W_k, c), _perm_half(W_v, c)], axis=1)
        for c in range(NC)
    ])
    pe = np.empty(64, np.int64)
    for m_ in range(2):
        for t_ in range(16):
            for b_ in range(2):
                pe[32 * m_ + 2 * t_ + b_] = 16 * (2 * m_ + b_) + t_
    pe = jnp.asarray(np.concatenate([pe, pe + 64]))
    w_e01_2 = jnp.stack([
        jnp.concatenate([_perm_half(W_e0, c), _perm_half(W_e1, c)],
                        axis=1)[:, pe]
        for c in range(NC)
    ])
    q2, kv2 = _project_nodes(x, w_q2, w_kv2)
    zeros = jnp.zeros((ROWS_PT, ACCW), jnp.float32)

    accs = []
    for s in range(NSPLIT):
        sl = slice(s * EH, (s + 1) * EH)
        sidx = edge_index[0, sl].reshape(NS, NCHUNK, B)
        didx = edge_index[1, sl].reshape(NS, NCHUNK, B)
        e01_2 = _project_edges(edge_feat, w_e01_2, s)
        accs.append(_sc_edge(sidx, didx, q2, kv2, e01_2, zeros))
    acc = accs

    sa = np.zeros((16, 128), np.float32)
    sb = np.zeros((16, 128), np.float32)
    for h in range(HL):
        for cc in range(16):
            sa[h, cc * 4 + h] = 1.0
            sb[h, 64 + cc * 4 + h] = 1.0
    sa = jnp.asarray(sa)
    sb = jnp.asarray(sb)

    perm = np.array([(HL * c + h) * C + cc
                     for c in range(NC) for cc in range(16)
                     for h in range(HL)])
    w1a = W1[:HID][jnp.asarray(perm)]
    w1b = W1[HID:]
    out = _mlp(acc, x, sa, sb, w1a, w1b, b1.reshape(1, -1),
               g1.reshape(1, -1), be1.reshape(1, -1), W2, b2.reshape(1, -1))
    return out


# speedup vs baseline: 1.6930x; 1.6930x over previous
import functools

import jax
import jax.numpy as jnp
import numpy as np
from jax import lax
from jax.experimental import pallas as pl
from jax.experimental.pallas import tpu as pltpu
from jax.experimental.pallas import tpu_sc as plsc

N = 10000
E = 320000
D = 128
H = 8
C = 16
DE = 16
HID = 128

NC = 2
NS = 16
HL = H // NC
HW = HL * C
NSPLIT = 4
EH = E // NSPLIT
EPS = EH // NS
B = 40
NCHUNK = EPS // B
NPAD = 10240
ROWS_PT = NPAD // NS
ACCW = 80

BN = 400
BE = 2000


def _proj_body(x_ref, wq_ref, wkv_ref, q_ref, kv_ref):
    x = x_ref[...]
    q_ref[0] = (x @ wq_ref[0]) * 0.25
    kv_ref[0] = x @ wkv_ref[0]


def _project_nodes(x, w_q2, w_kv2):
    return pl.pallas_call(
        _proj_body,
        grid=(NC, N // BN),
        in_specs=[
            pl.BlockSpec((BN, D), lambda c, i: (i, 0)),
            pl.BlockSpec((1, D, HW), lambda c, i: (c, 0, 0)),
            pl.BlockSpec((1, D, 2 * HW), lambda c, i: (c, 0, 0)),
        ],
        out_specs=[
            pl.BlockSpec((1, BN, HW), lambda c, i: (c, i, 0)),
            pl.BlockSpec((1, BN, 2 * HW), lambda c, i: (c, i, 0)),
        ],
        out_shape=[
            jax.ShapeDtypeStruct((NC, N, HW), jnp.float32),
            jax.ShapeDtypeStruct((NC, N, 2 * HW), jnp.float32),
        ],
    )(x, w_q2, w_kv2)


def _e01_body(ef_ref, we_ref, out_ref):
    out_ref[0] = jnp.tanh(ef_ref[...] @ we_ref[0])


def _project_edges(edge_feat, w_e01_2, s):
    nb = EH // BE
    return pl.pallas_call(
        _e01_body,
        grid=(NC, nb),
        in_specs=[
            pl.BlockSpec((BE, DE), lambda c, i: (s * nb + i, 0)),
            pl.BlockSpec((1, DE, 2 * HW), lambda c, i: (c, 0, 0)),
        ],
        out_specs=pl.BlockSpec((1, BE, 2 * HW), lambda c, i: (c, i, 0)),
        out_shape=jax.ShapeDtypeStruct((NC, EH, 2 * HW), jnp.float32),
    )(edge_feat, w_e01_2)


def _sc_edge_body(sidx_hbm, didx_hbm, q_hbm, kv_hbm, e01_hbm,
                  zeros_hbm, out_hbm, sidx_v, didx_v, qi_v, kv_v,
                  e01_v, msg_v, acc_sh, gsem0, gsem1, ssem0, ssem1):
    cid = lax.axis_index("c")
    sid = lax.axis_index("s")
    gsem = (gsem0, gsem1)
    ssem = (ssem0, ssem1)
    q_t = q_hbm.at[cid]
    kv_t = kv_hbm.at[cid]

    cpi0 = pltpu.async_copy(sidx_hbm.at[sid], sidx_v, gsem[0])
    cpi1 = pltpu.async_copy(didx_hbm.at[sid], didx_v, gsem[0])
    pltpu.sync_copy(zeros_hbm, acc_sh.at[pl.ds(sid * ROWS_PT, ROWS_PT)])
    cpi0.wait()
    cpi1.wait()
    plsc.subcore_barrier()

    ebase = sid * EPS
    iota = lax.iota(jnp.int32, 16)
    _gdn = lax.GatherDimensionNumbers(offset_dims=(),
                                      collapsed_slice_dims=(0,),
                                      start_index_map=(0,))

    def _lanes(x, idx):
        return lax.gather(x, idx.reshape(16, 1), _gdn, (1,),
                          mode=lax.GatherScatterMode.PROMISE_IN_BOUNDS,
                          unique_indices=True)

    def prefetch(j, b):
        eb = ebase + j * B
        cp0 = pltpu.async_copy(q_t.at[didx_v.at[j]], qi_v.at[b], gsem[b])
        cp1 = pltpu.async_copy(kv_t.at[sidx_v.at[j]], kv_v.at[b], gsem[b])
        cp2 = pltpu.async_copy(e01_hbm.at[cid, pl.ds(eb, B)], e01_v.at[b],
                               gsem[b])
        return cp0, cp1, cp2

    def wait_prefetch(b):
        pltpu.make_async_copy(q_t.at[didx_v.at[0]], qi_v.at[b],
                              gsem[b]).wait()
        pltpu.make_async_copy(kv_t.at[sidx_v.at[0]], kv_v.at[b],
                              gsem[b]).wait()
        pltpu.make_async_copy(e01_hbm.at[cid, pl.ds(ebase, B)], e01_v.at[b],
                              gsem[b]).wait()

    def wait_scatter(b):
        pltpu.make_async_copy(msg_v.at[b], acc_sh.at[didx_v.at[0]],
                              ssem[b]).wait()

    def compute(j, b, first):
        wait_prefetch(b)
        qi = qi_v.at[b]
        kv = kv_v.at[b]
        e01 = e01_v.at[b]
        msg = msg_v.at[b]

        @pl.when(jnp.logical_not(first))
        def _():
            wait_scatter(b)

        perm8 = iota ^ 8
        perm4 = iota ^ 4
        lt4 = iota < 4

        def edge(i, exv_prev):
            ia = jnp.minimum(i, B - 1)
            im = jnp.maximum(i - 1, 0)
            qs = [qi[ia, pl.ds(16 * k, 16)] for k in range(4)]
            ks = [kv[ia, pl.ds(16 * k, 16)] for k in range(4)]
            e0s = [e01[ia, pl.ds(16 * k, 16)] for k in range(4)]
            vs = [kv[im, pl.ds(HW + 16 * k, 16)] for k in range(4)]
            e1s = [e01[im, pl.ds(HW + 16 * k, 16)] for k in range(4)]
            ts = [qs[k] * ks[k] * e0s[k] for k in range(4)]
            ms = [vs[k] * e1s[k] * exv_prev for k in range(4)]
            t = (ts[0] + ts[1]) + (ts[2] + ts[3])
            t = t + _lanes(t, perm8)
            t = t + _lanes(t, perm4)
            exv = jnp.exp(t)
            for k in range(4):
                msg[im, pl.ds(16 * k, 16)] = ms[k]
            msg[im, pl.ds(HW, 16)] = jnp.where(lt4, exv_prev, 0.0)
            return exv

        lax.fori_loop(0, B + 1, edge, jnp.zeros((16,), jnp.float32))
        pltpu.async_copy(msg, acc_sh.at[didx_v.at[j]], ssem[b], add=True)

    prefetch(0, 0)

    def outer(t, carry):
        j0 = 2 * t
        prefetch(j0 + 1, 1)
        compute(j0, 0, t == 0)
        prefetch(jnp.minimum(j0 + 2, NCHUNK - 1), 0)
        compute(j0 + 1, 1, t == 0)
        return carry

    lax.fori_loop(0, (NCHUNK - 1) // 2, outer, 0)
    compute(NCHUNK - 1, 0, False)
    wait_scatter(1)
    wait_scatter(0)
    plsc.subcore_barrier()

    pltpu.sync_copy(acc_sh.at[pl.ds(sid * ROWS_PT, ROWS_PT)],
                    out_hbm.at[cid, pl.ds(sid * ROWS_PT, ROWS_PT)])


def _sc_edge(sidx, didx, q2, kv2, e01_2, zeros):
    mesh = plsc.VectorSubcoreMesh(core_axis_name="c", subcore_axis_name="s")
    f = functools.partial(
        pl.kernel,
        out_type=jax.ShapeDtypeStruct((NC, NPAD, ACCW), jnp.float32),
        mesh=mesh,
        compiler_params=pltpu.CompilerParams(use_tc_tiling_on_sc=False,
                                             needs_layout_passes=False),
        scratch_types=[
            pltpu.VMEM((NCHUNK, B), jnp.int32),
            pltpu.VMEM((NCHUNK, B), jnp.int32),
            pltpu.VMEM((2, B, HW), jnp.float32),
            pltpu.VMEM((2, B, 2 * HW), jnp.float32),
            pltpu.VMEM((2, B, 2 * HW), jnp.float32),
            pltpu.VMEM((2, B, ACCW), jnp.float32),
            pltpu.VMEM_SHARED((NPAD, ACCW), jnp.float32),
            pltpu.SemaphoreType.DMA,
            pltpu.SemaphoreType.DMA,
            pltpu.SemaphoreType.DMA,
            pltpu.SemaphoreType.DMA,
        ],
    )(_sc_edge_body)
    return f(sidx, didx, q2, kv2, e01_2, zeros)


def _mlp_body(*refs):
    acc_refs = refs[:2 * NSPLIT]
    (x_ref, sa_ref, sb_ref, w1a_ref, w1b_ref, b1_ref, g1_ref, be1_ref,
     w2_ref, b2_ref, out_ref) = refs[2 * NSPLIT:]
    acca = acc_refs[0][0]
    accb = acc_refs[1][0]
    for s in range(1, NSPLIT):
        acca = acca + acc_refs[2 * s][0]
        accb = accb + acc_refs[2 * s + 1][0]
    att = jnp.concatenate([acca[:, :HW], accb[:, :HW]], axis=1)
    den = acca[:, HW:] @ sa_ref[...] + accb[:, HW:] @ sb_ref[...]
    att = att / (den + 1e-16)
    h = att @ w1a_ref[...] + x_ref[...] @ w1b_ref[...] + b1_ref[...]
    mu = jnp.mean(h, axis=1, keepdims=True)
    var = jnp.mean((h - mu) ** 2, axis=1, keepdims=True)
    h = (h - mu) / jnp.sqrt(var + 1e-5) * g1_ref[...] + be1_ref[...]
    h = jnp.maximum(h, 0.0)
    out_ref[...] = h @ w2_ref[...] + b2_ref[...]


def _mlp(accs, x, sa, sb, w1a, w1b, b1, g1, be1, w2, b2):
    full = lambda shape: pl.BlockSpec(shape, lambda i: (0,) * len(shape))
    acc_specs = []
    acc_args = []
    for a in accs:
        acc_specs.append(pl.BlockSpec((1, BN, ACCW), lambda i: (0, i, 0)))
        acc_specs.append(pl.BlockSpec((1, BN, ACCW), lambda i: (1, i, 0)))
        acc_args.extend([a, a])
    return pl.pallas_call(
        _mlp_body,
        grid=(N // BN,),
        in_specs=acc_specs + [
            pl.BlockSpec((BN, D), lambda i: (i, 0)),
            full((16, 128)),
            full((16, 128)),
            full((HID, HID)),
            full((HID, HID)),
            full((1, HID)),
            full((1, HID)),
            full((1, HID)),
            full((HID, HID)),
            full((1, HID)),
        ],
        out_specs=pl.BlockSpec((BN, HID), lambda i: (i, 0)),
        out_shape=jax.ShapeDtypeStruct((N, HID), jnp.float32),
    )(*acc_args, x, sa, sb, w1a, w1b, b1, g1, be1, w2, b2)


def _perm_half(w, c):
    return w.reshape(-1, H, C)[:, HL * c:HL * (c + 1), :].transpose(
        0, 2, 1).reshape(-1, HW)


def kernel(x, edge_feat, edge_index, W_q, W_k, W_v, W_e0, W_e1, W1, b1, g1,
           be1, W2, b2):
    w_q2 = jnp.stack([_perm_half(W_q, c) for c in range(NC)])
    w_kv2 = jnp.stack([
        jnp.concatenate([_perm_half(W_k, c), _perm_half(W_v, c)], axis=1)
        for c in range(NC)
    ])
    w_e01_2 = jnp.stack([
        jnp.concatenate([_perm_half(W_e0, c), _perm_half(W_e1, c)], axis=1)
        for c in range(NC)
    ])
    q2, kv2 = _project_nodes(x, w_q2, w_kv2)
    zeros = jnp.zeros((ROWS_PT, ACCW), jnp.float32)

    accs = []
    for s in range(NSPLIT):
        sl = slice(s * EH, (s + 1) * EH)
        sidx = edge_index[0, sl].reshape(NS, NCHUNK, B)
        didx = edge_index[1, sl].reshape(NS, NCHUNK, B)
        e01_2 = _project_edges(edge_feat, w_e01_2, s)
        accs.append(_sc_edge(sidx, didx, q2, kv2, e01_2, zeros))
    acc = accs

    sa = np.zeros((16, 128), np.float32)
    sb = np.zeros((16, 128), np.float32)
    for h in range(HL):
        for cc in range(16):
            sa[h, cc * 4 + h] = 1.0
            sb[h, 64 + cc * 4 + h] = 1.0
    sa = jnp.asarray(sa)
    sb = jnp.asarray(sb)

    perm = np.array([(HL * c + h) * C + cc
                     for c in range(NC) for cc in range(16)
                     for h in range(HL)])
    w1a = W1[:HID][jnp.asarray(perm)]
    w1b = W1[HID:]
    out = _mlp(acc, x, sa, sb, w1a, w1b, b1.reshape(1, -1),
               g1.reshape(1, -1), be1.reshape(1, -1), W2, b2.reshape(1, -1))
    return out


# speedup vs baseline: 1.7615x; 1.0405x over previous
import functools

import jax
import jax.numpy as jnp
import numpy as np
from jax import lax
from jax.experimental import pallas as pl
from jax.experimental.pallas import tpu as pltpu
from jax.experimental.pallas import tpu_sc as plsc

N = 10000
E = 320000
D = 128
H = 8
C = 16
DE = 16
HID = 128

NC = 2
NS = 16
HL = H // NC
HW = HL * C
NSPLIT = 4
EH = E // NSPLIT
EPS = EH // NS
B = 40
NCHUNK = EPS // B
NPAD = 10240
ROWS_PT = NPAD // NS
ACCW = 80

BN = 400
BE = 2000


def _proj_body(x_ref, wq_ref, wkv_ref, q_ref, kv_ref):
    x = x_ref[...]
    q_ref[0] = (x @ wq_ref[0]) * 0.25
    kv_ref[0] = x @ wkv_ref[0]


def _project_nodes(x, w_q2, w_kv2):
    return pl.pallas_call(
        _proj_body,
        grid=(NC, N // BN),
        in_specs=[
            pl.BlockSpec((BN, D), lambda c, i: (i, 0)),
            pl.BlockSpec((1, D, HW), lambda c, i: (c, 0, 0)),
            pl.BlockSpec((1, D, 2 * HW), lambda c, i: (c, 0, 0)),
        ],
        out_specs=[
            pl.BlockSpec((1, BN, HW), lambda c, i: (c, i, 0)),
            pl.BlockSpec((1, BN, 2 * HW), lambda c, i: (c, i, 0)),
        ],
        out_shape=[
            jax.ShapeDtypeStruct((NC, N, HW), jnp.float32),
            jax.ShapeDtypeStruct((NC, N, 2 * HW), jnp.float32),
        ],
    )(x, w_q2, w_kv2)


def _e01_body(ef_ref, we_ref, out_ref):
    out_ref[0] = jnp.tanh(ef_ref[...] @ we_ref[0])


def _project_edges(edge_feat, w_e01_2):
    return pl.pallas_call(
        _e01_body,
        grid=(NC, EH // BE),
        in_specs=[
            pl.BlockSpec((BE, DE), lambda c, i: (i, 0)),
            pl.BlockSpec((1, DE, 2 * HW), lambda c, i: (c, 0, 0)),
        ],
        out_specs=pl.BlockSpec((1, BE, 2 * HW), lambda c, i: (c, i, 0)),
        out_shape=jax.ShapeDtypeStruct((NC, EH, 2 * HW), jnp.float32),
    )(edge_feat, w_e01_2)


def _sc_edge_body(sidx_hbm, didx_hbm, q_hbm, kv_hbm, e01_hbm,
                  zeros_hbm, out_hbm, sidx_v, didx_v, qi_v, kv_v,
                  e01_v, msg_v, acc_sh, gsem0, gsem1, ssem0, ssem1):
    cid = lax.axis_index("c")
    sid = lax.axis_index("s")
    gsem = (gsem0, gsem1)
    ssem = (ssem0, ssem1)
    q_t = q_hbm.at[cid]
    kv_t = kv_hbm.at[cid]

    cpi0 = pltpu.async_copy(sidx_hbm.at[sid], sidx_v, gsem[0])
    cpi1 = pltpu.async_copy(didx_hbm.at[sid], didx_v, gsem[0])
    pltpu.sync_copy(zeros_hbm, acc_sh.at[pl.ds(sid * ROWS_PT, ROWS_PT)])
    cpi0.wait()
    cpi1.wait()
    plsc.subcore_barrier()

    ebase = sid * EPS
    iota = lax.iota(jnp.int32, 16)
    _gdn = lax.GatherDimensionNumbers(offset_dims=(),
                                      collapsed_slice_dims=(0,),
                                      start_index_map=(0,))

    def _lanes(x, idx):
        return lax.gather(x, idx.reshape(16, 1), _gdn, (1,),
                          mode=lax.GatherScatterMode.PROMISE_IN_BOUNDS,
                          unique_indices=True)

    def prefetch(j, b):
        eb = ebase + j * B
        cp0 = pltpu.async_copy(q_t.at[didx_v.at[j]], qi_v.at[b], gsem[b])
        cp1 = pltpu.async_copy(kv_t.at[sidx_v.at[j]], kv_v.at[b], gsem[b])
        cp2 = pltpu.async_copy(e01_hbm.at[cid, pl.ds(eb, B)], e01_v.at[b],
                               gsem[b])
        return cp0, cp1, cp2

    def wait_prefetch(b):
        pltpu.make_async_copy(q_t.at[didx_v.at[0]], qi_v.at[b],
                              gsem[b]).wait()
        pltpu.make_async_copy(kv_t.at[sidx_v.at[0]], kv_v.at[b],
                              gsem[b]).wait()
        pltpu.make_async_copy(e01_hbm.at[cid, pl.ds(ebase, B)], e01_v.at[b],
                              gsem[b]).wait()

    def wait_scatter(b):
        pltpu.make_async_copy(msg_v.at[b], acc_sh.at[didx_v.at[0]],
                              ssem[b]).wait()

    def compute(j, b, first):
        wait_prefetch(b)
        qi = qi_v.at[b]
        kv = kv_v.at[b]
        e01 = e01_v.at[b]
        msg = msg_v.at[b]

        @pl.when(jnp.logical_not(first))
        def _():
            wait_scatter(b)

        perm8 = iota ^ 8
        perm4 = iota ^ 4
        lt4 = iota < 4

        def edge(i, exv_prev):
            ia = jnp.minimum(i, B - 1)
            im = jnp.maximum(i - 1, 0)
            qs = [qi[ia, pl.ds(16 * k, 16)] for k in range(4)]
            ks = [kv[ia, pl.ds(16 * k, 16)] for k in range(4)]
            e0s = [e01[ia, pl.ds(16 * k, 16)] for k in range(4)]
            vs = [kv[im, pl.ds(HW + 16 * k, 16)] for k in range(4)]
            e1s = [e01[im, pl.ds(HW + 16 * k, 16)] for k in range(4)]
            ts = [qs[k] * ks[k] * e0s[k] for k in range(4)]
            ms = [vs[k] * e1s[k] * exv_prev for k in range(4)]
            t = (ts[0] + ts[1]) + (ts[2] + ts[3])
            t = t + _lanes(t, perm8)
            t = t + _lanes(t, perm4)
            exv = jnp.exp(t)
            for k in range(4):
                msg[im, pl.ds(16 * k, 16)] = ms[k]
            msg[im, pl.ds(HW, 16)] = jnp.where(lt4, exv_prev, 0.0)
            return exv

        lax.fori_loop(0, B + 1, edge, jnp.zeros((16,), jnp.float32))
        pltpu.async_copy(msg, acc_sh.at[didx_v.at[j]], ssem[b], add=True)

    prefetch(0, 0)

    def outer(t, carry):
        j0 = 2 * t
        prefetch(j0 + 1, 1)
        compute(j0, 0, t == 0)
        prefetch(jnp.minimum(j0 + 2, NCHUNK - 1), 0)
        compute(j0 + 1, 1, t == 0)
        return carry

    lax.fori_loop(0, (NCHUNK - 1) // 2, outer, 0)
    compute(NCHUNK - 1, 0, False)
    wait_scatter(1)
    wait_scatter(0)
    plsc.subcore_barrier()

    pltpu.sync_copy(acc_sh.at[pl.ds(sid * ROWS_PT, ROWS_PT)],
                    out_hbm.at[cid, pl.ds(sid * ROWS_PT, ROWS_PT)])


def _sc_edge(sidx, didx, q2, kv2, e01_2, zeros):
    mesh = plsc.VectorSubcoreMesh(core_axis_name="c", subcore_axis_name="s")
    f = functools.partial(
        pl.kernel,
        out_type=jax.ShapeDtypeStruct((NC, NPAD, ACCW), jnp.float32),
        mesh=mesh,
        compiler_params=pltpu.CompilerParams(use_tc_tiling_on_sc=False,
                                             needs_layout_passes=False),
        scratch_types=[
            pltpu.VMEM((NCHUNK, B), jnp.int32),
            pltpu.VMEM((NCHUNK, B), jnp.int32),
            pltpu.VMEM((2, B, HW), jnp.float32),
            pltpu.VMEM((2, B, 2 * HW), jnp.float32),
            pltpu.VMEM((2, B, 2 * HW), jnp.float32),
            pltpu.VMEM((2, B, ACCW), jnp.float32),
            pltpu.VMEM_SHARED((NPAD, ACCW), jnp.float32),
            pltpu.SemaphoreType.DMA,
            pltpu.SemaphoreType.DMA,
            pltpu.SemaphoreType.DMA,
            pltpu.SemaphoreType.DMA,
        ],
    )(_sc_edge_body)
    return f(sidx, didx, q2, kv2, e01_2, zeros)


def _mlp_body(*refs):
    acc_refs = refs[:2 * NSPLIT]
    (x_ref, sa_ref, sb_ref, w1a_ref, w1b_ref, b1_ref, g1_ref, be1_ref,
     w2_ref, b2_ref, out_ref) = refs[2 * NSPLIT:]
    acca = acc_refs[0][0]
    accb = acc_refs[1][0]
    for s in range(1, NSPLIT):
        acca = acca + acc_refs[2 * s][0]
        accb = accb + acc_refs[2 * s + 1][0]
    att = jnp.concatenate([acca[:, :HW], accb[:, :HW]], axis=1)
    den = acca[:, HW:] @ sa_ref[...] + accb[:, HW:] @ sb_ref[...]
    att = att / (den + 1e-16)
    h = att @ w1a_ref[...] + x_ref[...] @ w1b_ref[...] + b1_ref[...]
    mu = jnp.mean(h, axis=1, keepdims=True)
    var = jnp.mean((h - mu) ** 2, axis=1, keepdims=True)
    h = (h - mu) / jnp.sqrt(var + 1e-5) * g1_ref[...] + be1_ref[...]
    h = jnp.maximum(h, 0.0)
    out_ref[...] = h @ w2_ref[...] + b2_ref[...]


def _mlp(accs, x, sa, sb, w1a, w1b, b1, g1, be1, w2, b2):
    full = lambda shape: pl.BlockSpec(shape, lambda i: (0,) * len(shape))
    acc_specs = []
    acc_args = []
    for a in accs:
        acc_specs.append(pl.BlockSpec((1, BN, ACCW), lambda i: (0, i, 0)))
        acc_specs.append(pl.BlockSpec((1, BN, ACCW), lambda i: (1, i, 0)))
        acc_args.extend([a, a])
    return pl.pallas_call(
        _mlp_body,
        grid=(N // BN,),
        in_specs=acc_specs + [
            pl.BlockSpec((BN, D), lambda i: (i, 0)),
            full((16, 128)),
            full((16, 128)),
            full((HID, HID)),
            full((HID, HID)),
            full((1, HID)),
            full((1, HID)),
            full((1, HID)),
            full((HID, HID)),
            full((1, HID)),
        ],
        out_specs=pl.BlockSpec((BN, HID), lambda i: (i, 0)),
        out_shape=jax.ShapeDtypeStruct((N, HID), jnp.float32),
    )(*acc_args, x, sa, sb, w1a, w1b, b1, g1, be1, w2, b2)


def _perm_half(w, c):
    return w.reshape(-1, H, C)[:, HL * c:HL * (c + 1), :].transpose(
        0, 2, 1).reshape(-1, HW)


def kernel(x, edge_feat, edge_index, W_q, W_k, W_v, W_e0, W_e1, W1, b1, g1,
           be1, W2, b2):
    w_q2 = jnp.stack([_perm_half(W_q, c) for c in range(NC)])
    w_kv2 = jnp.stack([
        jnp.concatenate([_perm_half(W_k, c), _perm_half(W_v, c)], axis=1)
        for c in range(NC)
    ])
    w_e01_2 = jnp.stack([
        jnp.concatenate([_perm_half(W_e0, c), _perm_half(W_e1, c)], axis=1)
        for c in range(NC)
    ])
    q2, kv2 = _project_nodes(x, w_q2, w_kv2)
    zeros = jnp.zeros((ROWS_PT, ACCW), jnp.float32)

    accs = []
    for s in range(NSPLIT):
        sl = slice(s * EH, (s + 1) * EH)
        sidx = edge_index[0, sl].reshape(NS, NCHUNK, B)
        didx = edge_index[1, sl].reshape(NS, NCHUNK, B)
        e01_2 = _project_edges(edge_feat[sl], w_e01_2)
        accs.append(_sc_edge(sidx, didx, q2, kv2, e01_2, zeros))
    acc = accs

    sa = np.zeros((16, 128), np.float32)
    sb = np.zeros((16, 128), np.float32)
    for h in range(HL):
        for cc in range(16):
            sa[h, cc * 4 + h] = 1.0
            sb[h, 64 + cc * 4 + h] = 1.0
    sa = jnp.asarray(sa)
    sb = jnp.asarray(sb)

    perm = np.array([(HL * c + h) * C + cc
                     for c in range(NC) for cc in range(16)
                     for h in range(HL)])
    w1a = W1[:HID][jnp.asarray(perm)]
    w1b = W1[HID:]
    out = _mlp(acc, x, sa, sb, w1a, w1b, b1.reshape(1, -1),
               g1.reshape(1, -1), be1.reshape(1, -1), W2, b2.reshape(1, -1))
    return out
